# baseline (speedup 1.0000x reference)
import functools

import jax
import jax.numpy as jnp
from jax import lax
from jax.experimental import pallas as pl
from jax.experimental.pallas import tpu as pltpu
from jax.experimental.pallas import tpu_sc as plsc

B = 16384
ED = 16
NC = 2
NS = 16
NW = NC * NS
BPW = B // NW
HALF = BPW // 2
GRP = 16


def _make_gather(nt):
    mesh = plsc.VectorSubcoreMesh(core_axis_name="c", subcore_axis_name="s")

    @functools.partial(
        pl.kernel,
        out_type=[jax.ShapeDtypeStruct((B, ED), jnp.float32)
                  for _ in range(nt)],
        mesh=mesh,
        scratch_types=[
            pltpu.VMEM((8, BPW + GRP), jnp.int32),
            pltpu.VMEM((HALF, ED), jnp.float32),
            pltpu.VMEM((HALF, ED), jnp.float32),
            pltpu.SemaphoreType.DMA,
            pltpu.SemaphoreType.DMA,
            pltpu.SemaphoreType.DMA,
        ],
    )
    def k(idx_hbm, *rest):
        tabs = list(rest[:nt])
        outs = list(rest[nt:2 * nt])
        idx_v, bufa, bufb, gsem, wsa, wsb = rest[2 * nt:]
        wid = lax.axis_index("s") * NC + lax.axis_index("c")
        base = wid * BPW
        bufs = [bufa, bufb]
        wsems = [wsa, wsb]
        for t in range(nt):
            pltpu.sync_copy(idx_hbm.at[pl.ds(t * B + base, BPW)],
                            idx_v.at[t, pl.ds(0, BPW)])
        pending = [None, None]
        for d in range(2 * nt):
            t, h = divmod(d, 2)
            p = d % 2
            if pending[p] is not None:
                pending[p].wait()
            tab = tabs[t]
            buf = bufs[p]

            def grp_body(g, _, t=t, h=h, tab=tab, buf=buf):
                j0 = h * HALF + g * GRP
                iv = idx_v[t, pl.ds(j0, GRP)]
                for kk in range(GRP):
                    pltpu.async_copy(tab.at[pl.ds(iv[kk], 1)],
                                     buf.at[pl.ds(g * GRP + kk, 1)], gsem)
                return 0

            lax.fori_loop(0, HALF // GRP, grp_body, 0)
            pltpu.make_async_copy(tab.at[pl.ds(0, HALF)], buf, gsem).wait()
            pending[p] = pltpu.async_copy(
                buf, outs[t].at[pl.ds(base + h * HALF, HALF)], wsems[p])
        pending[0].wait()
        pending[1].wait()

    return k


def _mlp_body(e0, e1, e2, e3, e4, e5, nf, w1, b1, w2, b2, out):
    num = jnp.dot(nf[:], w1[:], preferred_element_type=jnp.float32) + b1[:]
    feats = jnp.concatenate([e0[:], e1[:], e2[:], e3[:], e4[:], e5[:], num],
                            axis=-1)
    acc = jnp.dot(feats, w2[:], preferred_element_type=jnp.float32) + b2[:]
    out[:] = jnp.maximum(acc, 0.0)


def _mlp(e_list, nf, w1, b1, w2, b2):
    BB = 2048
    grid = (B // BB,)
    espec = pl.BlockSpec((BB, ED), lambda i: (i, 0))
    return pl.pallas_call(
        _mlp_body,
        grid=grid,
        in_specs=[espec] * 6 + [
            pl.BlockSpec((BB, 6), lambda i: (i, 0)),
            pl.BlockSpec((6, 20), lambda i: (0, 0)),
            pl.BlockSpec((1, 20), lambda i: (0, 0)),
            pl.BlockSpec((116, 64), lambda i: (0, 0)),
            pl.BlockSpec((1, 64), lambda i: (0, 0)),
        ],
        out_specs=pl.BlockSpec((BB, 64), lambda i: (i, 0)),
        out_shape=jax.ShapeDtypeStruct((B, 64), jnp.float32),
        compiler_params=pltpu.CompilerParams(
            dimension_semantics=("parallel",),
        ),
    )(*e_list, nf, w1, b1, w2, b2)


def kernel(model_name, pretrained_dataset, model_type, model_owner,
           model_architecture, model_task, numeric_features,
           T_name, T_ds, T_type, T_owner, T_arch, T_task, W1, b1, W2, b2):
    idx5 = jnp.stack([
        pretrained_dataset.astype(jnp.int32),
        model_type.astype(jnp.int32),
        model_owner.astype(jnp.int32),
        model_architecture.astype(jnp.int32),
        model_task.astype(jnp.int32),
    ], axis=0).reshape(5 * B)
    idx1 = model_name.astype(jnp.int32)
    e_rest = _make_gather(5)(idx5, T_ds, T_type, T_owner, T_arch, T_task)
    e_name = _make_gather(1)(idx1, T_name)
    e = [e_name[0]] + list(e_rest)
    return _mlp(e, numeric_features,
                W1, b1.reshape(1, 20), W2, b2.reshape(1, 64))
